# baseline (speedup 1.0000x reference)
import functools

import jax
import jax.numpy as jnp
from jax import lax
from jax.experimental import pallas as pl
from jax.experimental.pallas import tpu as pltpu
from jax.experimental.pallas import tpu_sc as plsc

B = 4
N = 4096
L = 16
NC = 2
NS = 16
NW = NC * NS
WPB = NW // B
CHUNK = N // WPB
NBLK = CHUNK // L


def _min_pass(a_xb, a_yb, a_2h, b_xb, b_yb, b_2h, o_v):
    init = tuple(jnp.full((L,), jnp.inf, jnp.float32) for _ in range(NBLK))

    def body(g, mins):
        av_x = a_xb[pl.ds(g * L, L)]
        av_y = a_yb[pl.ds(g * L, L)]
        av_2 = a_2h[pl.ds(g * L, L)]
        for j in range(L):
            ax = jnp.full((L,), av_x[j], jnp.float32)
            ay = jnp.full((L,), av_y[j], jnp.float32)
            a2 = jnp.full((L,), av_2[j], jnp.float32)
            out = []
            for k in range(NBLK):
                qx = b_xb[pl.ds(k * L, L)]
                qy = b_yb[pl.ds(k * L, L)]
                h = a2 - ax * qx - ay * qy
                out.append(jnp.minimum(mins[k], h))
            mins = tuple(out)
        return mins

    mins = lax.fori_loop(0, N // L, body, init)
    for k in range(NBLK):
        o_v[pl.ds(k * L, L)] = mins[k] + b_2h[pl.ds(k * L, L)]


def _sc_body(uxb, uyb, u2h, vxb, vyb, v2h, u2_out, v2_out,
             a_x, a_y, a_2, b_x, b_y, b_2, o_v):
    c = lax.axis_index("c")
    s = lax.axis_index("s")
    wid = s * NC + c
    b = wid // WPB
    base = (wid % WPB) * CHUNK

    pltpu.sync_copy(uxb.at[b], a_x)
    pltpu.sync_copy(uyb.at[b], a_y)
    pltpu.sync_copy(u2h.at[b], a_2)
    pltpu.sync_copy(vxb.at[b, pl.ds(base, CHUNK)], b_x)
    pltpu.sync_copy(vyb.at[b, pl.ds(base, CHUNK)], b_y)
    pltpu.sync_copy(v2h.at[b, pl.ds(base, CHUNK)], b_2)
    _min_pass(a_x, a_y, a_2, b_x, b_y, b_2, o_v)
    pltpu.sync_copy(o_v, v2_out.at[b, pl.ds(base, CHUNK)])

    pltpu.sync_copy(vxb.at[b], a_x)
    pltpu.sync_copy(vyb.at[b], a_y)
    pltpu.sync_copy(v2h.at[b], a_2)
    pltpu.sync_copy(uxb.at[b, pl.ds(base, CHUNK)], b_x)
    pltpu.sync_copy(uyb.at[b, pl.ds(base, CHUNK)], b_y)
    pltpu.sync_copy(u2h.at[b, pl.ds(base, CHUNK)], b_2)
    _min_pass(a_x, a_y, a_2, b_x, b_y, b_2, o_v)
    pltpu.sync_copy(o_v, u2_out.at[b, pl.ds(base, CHUNK)])


_sc_minima = pl.kernel(
    _sc_body,
    out_type=(
        jax.ShapeDtypeStruct((B, N), jnp.float32),
        jax.ShapeDtypeStruct((B, N), jnp.float32),
    ),
    mesh=plsc.VectorSubcoreMesh(core_axis_name="c", subcore_axis_name="s"),
    scratch_types=[
        pltpu.VMEM((N,), jnp.float32),
        pltpu.VMEM((N,), jnp.float32),
        pltpu.VMEM((N,), jnp.float32),
        pltpu.VMEM((CHUNK,), jnp.float32),
        pltpu.VMEM((CHUNK,), jnp.float32),
        pltpu.VMEM((CHUNK,), jnp.float32),
        pltpu.VMEM((CHUNK,), jnp.float32),
    ],
)


def _finish_body(u2_ref, v2_ref, o_ref):
    su = jnp.sum(jnp.sqrt(jnp.maximum(2.0 * u2_ref[...], 0.0)))
    sv = jnp.sum(jnp.sqrt(jnp.maximum(2.0 * v2_ref[...], 0.0)))
    o_ref[0, 0] = (su + sv) * (1.0 / (2.0 * B * N))


_finish = pl.pallas_call(
    _finish_body,
    out_shape=jax.ShapeDtypeStruct((1, 1), jnp.float32),
    out_specs=pl.BlockSpec(memory_space=pltpu.SMEM),
)


@jax.jit
def kernel(u_, v_):
    ux = u_[:, :, 0]
    uy = u_[:, :, 1]
    vx = v_[:, :, 0]
    vy = v_[:, :, 1]
    ub16 = lax.optimization_barrier((ux.astype(jnp.bfloat16),
                                     uy.astype(jnp.bfloat16),
                                     vx.astype(jnp.bfloat16),
                                     vy.astype(jnp.bfloat16)))
    uxb = ub16[0].astype(jnp.float32)
    uyb = ub16[1].astype(jnp.float32)
    vxb = ub16[2].astype(jnp.float32)
    vyb = ub16[3].astype(jnp.float32)
    u2h = 0.5 * (ux * ux + uy * uy)
    v2h = 0.5 * (vx * vx + vy * vy)
    u2g, v2g = _sc_minima(uxb, uyb, u2h, vxb, vyb, v2h)
    return _finish(u2g, v2g)[0, 0]


# speedup vs baseline: 12.4162x; 12.4162x over previous
import functools

import jax
import jax.numpy as jnp
from jax import lax
from jax.experimental import pallas as pl
from jax.experimental.pallas import tpu as pltpu
from jax.experimental.pallas import tpu_sc as plsc

B = 4
N = 4096
M = 4096
L = 16
NC = 2
NS = 16
NW = NC * NS
S = 512
MT = M - S
WPB = NW // B
CH = S // WPB
KB = CH // L
RT = 512


def _sc_body(uxb, uyb, u2h, vxs, vys, v2s, row_out, col_out,
             a_x, a_y, a_2, b_x, b_y, b_2, rbuf, o_c):
    c = lax.axis_index("c")
    s = lax.axis_index("s")
    wid = s * NC + c
    b = wid // WPB
    base = (wid % WPB) * CH

    pltpu.sync_copy(uxb.at[b], a_x)
    pltpu.sync_copy(uyb.at[b], a_y)
    pltpu.sync_copy(u2h.at[b], a_2)
    pltpu.sync_copy(vxs.at[b, pl.ds(base, CH)], b_x)
    pltpu.sync_copy(vys.at[b, pl.ds(base, CH)], b_y)
    pltpu.sync_copy(v2s.at[b, pl.ds(base, CH)], b_2)

    qx = [b_x[pl.ds(k * L, L)] for k in range(KB)]
    qy = [b_y[pl.ds(k * L, L)] for k in range(KB)]
    w2 = [b_2[pl.ds(k * L, L)] for k in range(KB)]

    init = tuple(jnp.full((L,), jnp.inf, jnp.float32) for _ in range(KB))

    def body(g, cm):
        av_x = a_x[pl.ds(g * L, L)]
        av_y = a_y[pl.ds(g * L, L)]
        av_2 = a_2[pl.ds(g * L, L)]
        for j in range(L):
            bx = jnp.full((L,), av_x[j], jnp.float32)
            by = jnp.full((L,), av_y[j], jnp.float32)
            b2 = jnp.full((L,), av_2[j], jnp.float32)
            cm_new = []
            racc = None
            for k in range(KB):
                q = bx * qx[k] + by * qy[k]
                cm_new.append(jnp.minimum(cm[k], b2 - q))
                rc = w2[k] - q
                racc = rc if racc is None else jnp.minimum(racc, rc)
            cm = tuple(cm_new)
            rbuf[pl.ds((g * L + j) * L, L)] = racc
        return cm

    cmins = lax.fori_loop(0, N // L, body, init)
    for k in range(KB):
        o_c[pl.ds(k * L, L)] = cmins[k] + w2[k]
    pltpu.sync_copy(rbuf, row_out.at[b, wid % WPB])
    pltpu.sync_copy(o_c, col_out.at[b, pl.ds(base, CH)])


_sc_stripe = pl.kernel(
    _sc_body,
    out_type=(
        jax.ShapeDtypeStruct((B, WPB, N * L), jnp.float32),
        jax.ShapeDtypeStruct((B, S), jnp.float32),
    ),
    mesh=plsc.VectorSubcoreMesh(core_axis_name="c", subcore_axis_name="s"),
    scratch_types=[
        pltpu.VMEM((N,), jnp.float32),
        pltpu.VMEM((N,), jnp.float32),
        pltpu.VMEM((N,), jnp.float32),
        pltpu.VMEM((CH,), jnp.float32),
        pltpu.VMEM((CH,), jnp.float32),
        pltpu.VMEM((CH,), jnp.float32),
        pltpu.VMEM((N * L,), jnp.float32),
        pltpu.VMEM((CH,), jnp.float32),
    ],
)


def _tc_body(ub_ref, vb_ref, u2h_ref, v2h_ref, row_ref, col_ref):
    it = pl.program_id(1)
    inner = lax.dot_general(
        ub_ref[0], vb_ref[0], (((1,), (0,)), ((), ())),
        preferred_element_type=jnp.float32)
    t = v2h_ref[0, 0][None, :] - inner
    row_ref[0, 0, :] = jnp.min(t, axis=1)
    colp = jnp.min(u2h_ref[0, 0][:, None] + t, axis=0)

    @pl.when(it == 0)
    def _():
        col_ref[0, 0, :] = colp

    @pl.when(it != 0)
    def _():
        col_ref[0, 0, :] = jnp.minimum(col_ref[0, 0, :], colp)


_tc_main = pl.pallas_call(
    _tc_body,
    grid=(B, N // RT),
    in_specs=[
        pl.BlockSpec((1, RT, 2), lambda b, i: (b, i, 0)),
        pl.BlockSpec((1, 2, MT), lambda b, i: (b, 0, 0)),
        pl.BlockSpec((1, 1, RT), lambda b, i: (b * (N // RT) + i, 0, 0)),
        pl.BlockSpec((1, 1, MT), lambda b, i: (b, 0, 0)),
    ],
    out_specs=[
        pl.BlockSpec((1, 1, RT), lambda b, i: (b * (N // RT) + i, 0, 0)),
        pl.BlockSpec((1, 1, MT), lambda b, i: (b, 0, 0)),
    ],
    out_shape=[
        jax.ShapeDtypeStruct((B * (N // RT), 1, RT), jnp.float32),
        jax.ShapeDtypeStruct((B, 1, MT), jnp.float32),
    ],
)


def _combine_body(tc_row_ref, sc_row_ref, u2h_ref, tc_col_ref, sc_col_ref,
                  o_ref):
    sc_row = jnp.min(sc_row_ref[...].reshape(B, WPB, N, L), axis=(1, 3))
    row_g = jnp.minimum(tc_row_ref[...], sc_row) + u2h_ref[...]
    sr = jnp.sum(jnp.sqrt(jnp.maximum(2.0 * row_g, 0.0)))
    sc = jnp.sum(jnp.sqrt(jnp.maximum(2.0 * tc_col_ref[...], 0.0)))
    sc = sc + jnp.sum(jnp.sqrt(jnp.maximum(2.0 * sc_col_ref[...], 0.0)))
    o_ref[0, 0] = (sr + sc) * (1.0 / (2.0 * B * N))


_combine = pl.pallas_call(
    _combine_body,
    out_shape=jax.ShapeDtypeStruct((1, 1), jnp.float32),
    out_specs=pl.BlockSpec(memory_space=pltpu.SMEM),
)


@jax.jit
def kernel(u_, v_):
    ux = u_[:, :, 0]
    uy = u_[:, :, 1]
    vx = v_[:, :, 0]
    vy = v_[:, :, 1]
    ub16, vb16 = lax.optimization_barrier(
        (u_.astype(jnp.bfloat16), v_.astype(jnp.bfloat16)))
    u2h = 0.5 * (ux * ux + uy * uy)
    v2h = 0.5 * (vx * vx + vy * vy)

    uxb = ub16[:, :, 0].astype(jnp.float32)
    uyb = ub16[:, :, 1].astype(jnp.float32)
    vxb = vb16[:, :, 0].astype(jnp.float32)
    vyb = vb16[:, :, 1].astype(jnp.float32)

    sc_row, sc_col = _sc_stripe(
        uxb, uyb, u2h,
        vxb[:, MT:], vyb[:, MT:], v2h[:, MT:])

    vbt = jnp.transpose(vb16, (0, 2, 1))[:, :, :MT]
    tc_row, tc_col = _tc_main(
        ub16,
        vbt,
        u2h.reshape(B * (N // RT), 1, RT),
        v2h[:, :MT].reshape(B, 1, MT),
    )
    tc_row = tc_row.reshape(B, N)
    tc_col = tc_col.reshape(B, MT)

    return _combine(tc_row, sc_row, u2h, tc_col, sc_col)[0, 0]
